# Initial kernel scaffold; baseline (speedup 1.0000x reference)
import jax
import jax.numpy as jnp
from jax.experimental import pallas as pl


def kernel(input_tensors):
    raise NotImplementedError("write your pallas kernel here")


# speedup vs baseline: 1.9619x; 1.9619x over previous
import functools
import jax
import jax.numpy as jnp
from jax import lax
from jax.experimental import pallas as pl
from jax.experimental.pallas import tpu as pltpu
from jax.experimental.pallas import tpu_sc as plsc

R = 1024
N = 32768
L = 16
NB = 256
NPASS = 4
VREGS = N // L
MIN32 = jnp.int32(-2147483648)


@functools.cache
def _build_sort_kernel():
    info = plsc.get_sparse_core_info()
    nw = info.num_cores * info.num_subcores
    assert R % nw == 0
    rows_per_w = R // nw
    mesh = plsc.VectorSubcoreMesh(core_axis_name="c", subcore_axis_name="s")

    @functools.partial(
        pl.kernel,
        out_type=[
            jax.ShapeDtypeStruct((R, N), jnp.int32),
            jax.ShapeDtypeStruct((R, N), jnp.int32),
        ],
        mesh=mesh,
        compiler_params=pltpu.CompilerParams(needs_layout_passes=False),
        scratch_types=[
            pltpu.VMEM((N,), jnp.int32),
            pltpu.VMEM((N,), jnp.int32),
            pltpu.VMEM((N,), jnp.int32),
            pltpu.VMEM((NB,), jnp.int32),
            pltpu.VMEM((NB,), jnp.int32),
        ],
    )
    def sort_kernel(xbits_hbm, vals_hbm, idx_hbm, kbuf, abuf, bbuf, hist, offs):
        wid = lax.axis_index("s") * info.num_cores + lax.axis_index("c")
        iota = lax.iota(jnp.int32, L)
        zeros = jnp.zeros((L,), jnp.int32)

        def row_body(r, _):
            row = wid * rows_per_w + r
            pltpu.sync_copy(xbits_hbm.at[row], bbuf)

            def clear_body(j, _):
                hist[pl.ds(j * L, L)] = zeros
                return 0

            lax.fori_loop(0, NB // L, clear_body, 0)

            def conv_body(v, _):
                b = bbuf[pl.ds(v * L, L)]
                m = jnp.where(b >= 0, b ^ MIN32, ~b)
                kbuf[pl.ds(v * L, L)] = m
                abuf[pl.ds(v * L, L)] = v * L + iota
                dig = m & 0xFF
                cnt, lastm = plsc.scan_count(dig)
                plsc.addupdate_scatter(hist, [dig], cnt, mask=lastm)
                return 0

            lax.fori_loop(0, VREGS, conv_body, 0)

            for p in range(NPASS):
                src = abuf if p % 2 == 0 else bbuf
                dst = bbuf if p % 2 == 0 else abuf

                def scan_body(j, carry):
                    h = hist[pl.ds(j * L, L)]
                    inc = plsc.cumsum(h)
                    offs[pl.ds(j * L, L)] = inc - h + carry
                    hist[pl.ds(j * L, L)] = zeros
                    return carry + jnp.sum(h)

                lax.fori_loop(0, NB // L, scan_body, jnp.int32(0))

                shift = jnp.int32(8 * p)
                nshift = jnp.int32(8 * (p + 1))

                def perm_body(v, _, p=p, shift=shift, nshift=nshift, src=src, dst=dst):
                    i16 = src[pl.ds(v * L, L)]
                    m = plsc.load_gather(kbuf, [i16])
                    dig = lax.shift_right_logical(m, shift) & 0xFF
                    cnt, lastm = plsc.scan_count(dig)
                    base = plsc.load_gather(offs, [dig])
                    plsc.store_scatter(dst, [base + cnt - 1], i16)
                    plsc.addupdate_scatter(offs, [dig], cnt, mask=lastm)
                    if p < NPASS - 1:
                        dig2 = lax.shift_right_logical(m, nshift) & 0xFF
                        cnt2, lastm2 = plsc.scan_count(dig2)
                        plsc.addupdate_scatter(hist, [dig2], cnt2, mask=lastm2)
                    return 0

                lax.fori_loop(0, VREGS, perm_body, 0)

            def fin_body(v, _):
                i16 = abuf[pl.ds(v * L, L)]
                m = plsc.load_gather(kbuf, [i16])
                bbuf[pl.ds(v * L, L)] = jnp.where(m < 0, m ^ MIN32, ~m)
                return 0

            lax.fori_loop(0, VREGS, fin_body, 0)
            pltpu.sync_copy(bbuf, vals_hbm.at[row])
            pltpu.sync_copy(abuf, idx_hbm.at[row])
            return 0

        lax.fori_loop(0, rows_per_w, row_body, 0)

    return sort_kernel


@jax.jit
def kernel(input_tensors):
    xbits = lax.bitcast_convert_type(input_tensors, jnp.int32)
    vbits, idx = _build_sort_kernel()(xbits)
    values = lax.bitcast_convert_type(vbits, jnp.float32)
    return (values, idx)


# speedup vs baseline: 2.5764x; 1.3132x over previous
import functools
import jax
import jax.numpy as jnp
from jax import lax
from jax.experimental import pallas as pl
from jax.experimental.pallas import tpu as pltpu
from jax.experimental.pallas import tpu_sc as plsc

R = 1024
N = 32768
L = 16
NB = 2048
VREGS = N // L
MIN32 = jnp.int32(-2147483648)
MASK11 = jnp.int32(0x7FF)


@functools.cache
def _build_sort_kernel():
    info = plsc.get_sparse_core_info()
    nw = info.num_cores * info.num_subcores
    assert R % nw == 0
    rows_per_w = R // nw
    mesh = plsc.VectorSubcoreMesh(core_axis_name="c", subcore_axis_name="s")

    @functools.partial(
        pl.kernel,
        out_type=[
            jax.ShapeDtypeStruct((R, N), jnp.int32),
            jax.ShapeDtypeStruct((R, N), jnp.int32),
        ],
        mesh=mesh,
        compiler_params=pltpu.CompilerParams(needs_layout_passes=False),
        scratch_types=[
            pltpu.VMEM((N,), jnp.int32),
            pltpu.VMEM((N,), jnp.int32),
            pltpu.VMEM((N,), jnp.int32),
            pltpu.VMEM((NB,), jnp.int32),
            pltpu.VMEM((NB,), jnp.int32),
        ],
    )
    def sort_kernel(xbits_hbm, vals_hbm, idx_hbm, kbuf, abuf, bbuf, hist, offs):
        wid = lax.axis_index("s") * info.num_cores + lax.axis_index("c")
        iota = lax.iota(jnp.int32, L)
        zeros = jnp.zeros((L,), jnp.int32)

        def clear_body(j, _):
            hist[pl.ds(j * L, L)] = zeros
            return 0

        lax.fori_loop(0, NB // L, clear_body, 0)

        def row_body(r, _):
            row = wid * rows_per_w + r
            pltpu.sync_copy(xbits_hbm.at[row], bbuf)

            def conv_body(v, _):
                b = bbuf[pl.ds(v * L, L)]
                m = jnp.where(b >= 0, b ^ MIN32, ~b)
                kbuf[pl.ds(v * L, L)] = m
                cnt, lastm = plsc.scan_count(m & MASK11)
                plsc.addupdate_scatter(hist, [m & MASK11], cnt, mask=lastm)
                return 0

            lax.fori_loop(0, VREGS, conv_body, 0)

            def make_scan_body():
                def scan_body(j, carry):
                    h = hist[pl.ds(j * L, L)]
                    inc = plsc.cumsum(h)
                    offs[pl.ds(j * L, L)] = inc - h + carry
                    hist[pl.ds(j * L, L)] = zeros
                    return carry + jnp.sum(h)

                return scan_body

            def permute(i16, m, shift, nshift, dst):
                dig = lax.shift_right_logical(m, shift) & MASK11 if shift else m & MASK11
                cnt, lastm = plsc.scan_count(dig)
                base = plsc.load_gather(offs, [dig])
                plsc.store_scatter(dst, [base + cnt], i16)
                plsc.addupdate_scatter(offs, [dig], cnt, mask=lastm)
                if nshift is not None:
                    dig2 = lax.shift_right_logical(m, nshift)
                    if nshift < 22:
                        dig2 = dig2 & MASK11
                    cnt2, lastm2 = plsc.scan_count(dig2)
                    plsc.addupdate_scatter(hist, [dig2], cnt2, mask=lastm2)

            lax.fori_loop(0, NB // L, make_scan_body(), jnp.int32(-1))

            def p0_body(v, _):
                m = kbuf[pl.ds(v * L, L)]
                permute(v * L + iota, m, 0, 11, abuf)
                return 0

            lax.fori_loop(0, VREGS, p0_body, 0)

            lax.fori_loop(0, NB // L, make_scan_body(), jnp.int32(-1))

            def p1_body(v, _):
                i16 = abuf[pl.ds(v * L, L)]
                permute(i16, plsc.load_gather(kbuf, [i16]), 11, 22, bbuf)
                return 0

            lax.fori_loop(0, VREGS, p1_body, 0)

            lax.fori_loop(0, NB // L, make_scan_body(), jnp.int32(-1))

            def p2_body(v, _):
                i16 = bbuf[pl.ds(v * L, L)]
                permute(i16, plsc.load_gather(kbuf, [i16]), 22, None, abuf)
                return 0

            lax.fori_loop(0, VREGS, p2_body, 0)

            def fin_body(v, _):
                i16 = abuf[pl.ds(v * L, L)]
                m = plsc.load_gather(kbuf, [i16])
                bbuf[pl.ds(v * L, L)] = jnp.where(m < 0, m ^ MIN32, ~m)
                return 0

            lax.fori_loop(0, VREGS, fin_body, 0)
            pltpu.sync_copy(bbuf, vals_hbm.at[row])
            pltpu.sync_copy(abuf, idx_hbm.at[row])
            return 0

        lax.fori_loop(0, rows_per_w, row_body, 0)

    return sort_kernel


@jax.jit
def kernel(input_tensors):
    xbits = lax.bitcast_convert_type(input_tensors, jnp.int32)
    vbits, idx = _build_sort_kernel()(xbits)
    values = lax.bitcast_convert_type(vbits, jnp.float32)
    return (values, idx)
